# Initial kernel scaffold; baseline (speedup 1.0000x reference)
import jax
import jax.numpy as jnp
from jax.experimental import pallas as pl


def kernel(inputs, weights, biases, responses, src_col, dst_idx):
    raise NotImplementedError("write your pallas kernel here")


# speedup vs baseline: 107.4793x; 107.4793x over previous
import jax
import jax.numpy as jnp
from jax.experimental import pallas as pl
from jax.experimental.pallas import tpu as pltpu

_N_IN = 64
_N_OUT = 32
_TILE = 1024


def _body(x_ref, w_ref, b_ref, r_ref, src_ref, dst_ref, o_ref, a_ref, m_ref):
    e = src_ref.shape[0]

    @pl.when(pl.program_id(0) == 0)
    def _build():
        dst = dst_ref[...]
        src = src_ref[...]
        d_oh = (jax.lax.broadcasted_iota(jnp.int32, (_N_OUT, e), 0) == dst
                ).astype(jnp.float32)
        s_oh = jnp.where(
            jax.lax.broadcasted_iota(jnp.int32, (e, _N_IN), 1) == src,
            w_ref[...], 0.0)
        a_ref[...] = jnp.dot(d_oh, s_oh, preferred_element_type=jnp.float32)
        cnt = jnp.sum(d_oh, axis=1)[None, :]
        m_ref[...] = (cnt > 0.0).astype(jnp.float32)

    z = b_ref[...] + r_ref[...] * jnp.dot(
        x_ref[...], a_ref[...].T, preferred_element_type=jnp.float32)
    o_ref[...] = jax.nn.sigmoid(z) * m_ref[...]


def kernel(inputs, weights, biases, responses, src_col, dst_idx):
    b, n_in = inputs.shape
    e = weights.shape[0]
    src2d = src_col.astype(jnp.int32).reshape(e, 1)
    dst2d = dst_idx.astype(jnp.int32).reshape(1, e)
    w2d = weights.reshape(e, 1)
    b2d = biases[:_N_OUT].reshape(1, _N_OUT)
    r2d = responses[:_N_OUT].reshape(1, _N_OUT)

    grid = (b // _TILE,)
    return pl.pallas_call(
        _body,
        grid=grid,
        in_specs=[
            pl.BlockSpec((_TILE, n_in), lambda i: (i, 0)),
            pl.BlockSpec((e, 1), lambda i: (0, 0)),
            pl.BlockSpec((1, _N_OUT), lambda i: (0, 0)),
            pl.BlockSpec((1, _N_OUT), lambda i: (0, 0)),
            pl.BlockSpec((e, 1), lambda i: (0, 0)),
            pl.BlockSpec((1, e), lambda i: (0, 0)),
        ],
        out_specs=pl.BlockSpec((_TILE, _N_OUT), lambda i: (i, 0)),
        out_shape=jax.ShapeDtypeStruct((b, _N_OUT), inputs.dtype),
        scratch_shapes=[
            pltpu.VMEM((_N_OUT, n_in), jnp.float32),
            pltpu.VMEM((1, _N_OUT), jnp.float32),
        ],
    )(inputs, w2d, b2d, r2d, src2d, dst2d)


# speedup vs baseline: 131.4506x; 1.2230x over previous
import jax
import jax.numpy as jnp
from jax.experimental import pallas as pl
from jax.experimental.pallas import tpu as pltpu

_N_IN = 64
_N_OUT = 32
_TILE = 4096


def _body(x_ref, w_ref, b_ref, r_ref, src_ref, dst_ref, o_ref, a_ref, m_ref):
    e = src_ref.shape[0]

    @pl.when(pl.program_id(0) == 0)
    def _build():
        dst = dst_ref[...]
        src = src_ref[...]
        d_oh = (jax.lax.broadcasted_iota(jnp.int32, (_N_OUT, e), 0) == dst
                ).astype(jnp.float32)
        s_oh = jnp.where(
            jax.lax.broadcasted_iota(jnp.int32, (e, _N_IN), 1) == src,
            w_ref[...], 0.0)
        a_ref[...] = jnp.dot(d_oh, s_oh, preferred_element_type=jnp.float32)
        cnt = jnp.sum(d_oh, axis=1)[None, :]
        m_ref[...] = (cnt > 0.0).astype(jnp.float32)

    z = b_ref[...] + r_ref[...] * jnp.dot(
        x_ref[...], a_ref[...].T, preferred_element_type=jnp.float32)
    o_ref[...] = jax.nn.sigmoid(z) * m_ref[...]


def kernel(inputs, weights, biases, responses, src_col, dst_idx):
    b, n_in = inputs.shape
    e = weights.shape[0]
    src2d = src_col.astype(jnp.int32).reshape(e, 1)
    dst2d = dst_idx.astype(jnp.int32).reshape(1, e)
    w2d = weights.reshape(e, 1)
    b2d = biases[:_N_OUT].reshape(1, _N_OUT)
    r2d = responses[:_N_OUT].reshape(1, _N_OUT)

    grid = (b // _TILE,)
    return pl.pallas_call(
        _body,
        grid=grid,
        in_specs=[
            pl.BlockSpec((_TILE, n_in), lambda i: (i, 0)),
            pl.BlockSpec((e, 1), lambda i: (0, 0)),
            pl.BlockSpec((1, _N_OUT), lambda i: (0, 0)),
            pl.BlockSpec((1, _N_OUT), lambda i: (0, 0)),
            pl.BlockSpec((e, 1), lambda i: (0, 0)),
            pl.BlockSpec((1, e), lambda i: (0, 0)),
        ],
        out_specs=pl.BlockSpec((_TILE, _N_OUT), lambda i: (i, 0)),
        out_shape=jax.ShapeDtypeStruct((b, _N_OUT), inputs.dtype),
        scratch_shapes=[
            pltpu.VMEM((_N_OUT, n_in), jnp.float32),
            pltpu.VMEM((1, _N_OUT), jnp.float32),
        ],
    )(inputs, w2d, b2d, r2d, src2d, dst2d)


# speedup vs baseline: 138.6959x; 1.0551x over previous
import jax
import jax.numpy as jnp
from jax.experimental import pallas as pl
from jax.experimental.pallas import tpu as pltpu

_N_IN = 64
_N_OUT = 32
_TILE = 8192


def _body(x_ref, w_ref, b_ref, r_ref, src_ref, dst_ref, o_ref, a_ref, m_ref):
    e = src_ref.shape[0]

    @pl.when(pl.program_id(0) == 0)
    def _build():
        dst = dst_ref[...]
        src = src_ref[...]
        d_oh = (jax.lax.broadcasted_iota(jnp.int32, (_N_OUT, e), 0) == dst
                ).astype(jnp.float32)
        s_oh = jnp.where(
            jax.lax.broadcasted_iota(jnp.int32, (e, _N_IN), 1) == src,
            w_ref[...], 0.0)
        a_ref[...] = jnp.dot(d_oh, s_oh, preferred_element_type=jnp.float32)
        cnt = jnp.sum(d_oh, axis=1)[None, :]
        m_ref[...] = (cnt > 0.0).astype(jnp.float32)

    z = b_ref[...] + r_ref[...] * jnp.dot(
        x_ref[...], a_ref[...].T, preferred_element_type=jnp.float32)
    o_ref[...] = jax.nn.sigmoid(z) * m_ref[...]


def kernel(inputs, weights, biases, responses, src_col, dst_idx):
    b, n_in = inputs.shape
    e = weights.shape[0]
    src2d = src_col.astype(jnp.int32).reshape(e, 1)
    dst2d = dst_idx.astype(jnp.int32).reshape(1, e)
    w2d = weights.reshape(e, 1)
    b2d = biases[:_N_OUT].reshape(1, _N_OUT)
    r2d = responses[:_N_OUT].reshape(1, _N_OUT)

    grid = (b // _TILE,)
    return pl.pallas_call(
        _body,
        grid=grid,
        in_specs=[
            pl.BlockSpec((_TILE, n_in), lambda i: (i, 0)),
            pl.BlockSpec((e, 1), lambda i: (0, 0)),
            pl.BlockSpec((1, _N_OUT), lambda i: (0, 0)),
            pl.BlockSpec((1, _N_OUT), lambda i: (0, 0)),
            pl.BlockSpec((e, 1), lambda i: (0, 0)),
            pl.BlockSpec((1, e), lambda i: (0, 0)),
        ],
        out_specs=pl.BlockSpec((_TILE, _N_OUT), lambda i: (i, 0)),
        out_shape=jax.ShapeDtypeStruct((b, _N_OUT), inputs.dtype),
        scratch_shapes=[
            pltpu.VMEM((_N_OUT, n_in), jnp.float32),
            pltpu.VMEM((1, _N_OUT), jnp.float32),
        ],
    )(inputs, w2d, b2d, r2d, src2d, dst2d)
